# baseline (speedup 1.0000x reference)
import functools
import math

import jax
import jax.numpy as jnp
from jax import lax
from jax.experimental import pallas as pl
from jax.experimental.pallas import tpu as pltpu
from jax.experimental.pallas import tpu_sc as plsc

N_V = 10000
N_E = 2500
NNZ = 320000
D_IN = 128
D_HID = 64
N_HEADS = 4
N_CLASSES = 40
NEG_SLOPE = 0.2

NV_PAD = 10112
NE_PAD = 2560
_NS = 16
_B = 128
C1 = 144
C2 = 128
C3 = 48


def _leaky(x):
    return jnp.where(x >= 0, x, NEG_SLOPE * x)


@functools.lru_cache(maxsize=None)
def _sc_pass(tn, cw, nch, nd_pad):
    zr = nd_pad // _NS
    mesh = plsc.VectorSubcoreMesh(core_axis_name="c", subcore_axis_name="s",
                                  num_cores=2, num_subcores=_NS)

    def body(table, cidx, zeros, out, ixa, ixb, r0, r1, acc,
             sg0, sg1, ss0, ss1, sxa, sxb):
        ci = lax.axis_index("c")
        si = lax.axis_index("s")
        pltpu.sync_copy(zeros, acc.at[pl.ds(si * zr, zr)])
        pltpu.sync_copy(cidx.at[ci, si, 0], ixa)
        pltpu.sync_copy(cidx.at[ci, si, 1], ixb)
        plsc.subcore_barrier()

        rbufs = (r0, r1)
        sgs = (sg0, sg1)
        sss = (ss0, ss1)

        def drain(sem):
            pltpu.make_async_copy(table.at[pl.ds(0, _B)], r0, sem).wait()

        pltpu.async_copy(table.at[ixa.at[0, 0]], r0, sg0)
        pltpu.async_copy(table.at[ixa.at[1, 0]], r1, sg1)

        def super_step(k, carry):
            for half, (ixc, ixn, sxc) in enumerate(
                    ((ixa, ixb, sxa), (ixb, ixa, sxb))):
                for t in (0, 1):
                    drain(sgs[t])
                    pltpu.async_copy(rbufs[t], acc.at[ixc.at[t, 1]],
                                     sss[t], add=True)
                for t in (0, 1):
                    drain(sss[t])
                    pltpu.async_copy(table.at[ixc.at[2 + t, 0]],
                                     rbufs[t], sgs[t])
                for t in (0, 1):
                    drain(sgs[t])
                    pltpu.async_copy(rbufs[t], acc.at[ixc.at[2 + t, 1]],
                                     sss[t], add=True)
                if half == 0:
                    @pl.when(k > 0)
                    def _():
                        pltpu.make_async_copy(
                            cidx.at[ci, si, 0], ixb, sxb).wait()

                    for t in (0, 1):
                        drain(sss[t])
                        pltpu.async_copy(table.at[ixn.at[t, 0]],
                                         rbufs[t], sgs[t])

                    @pl.when(k < nch // 8 - 1)
                    def _():
                        pltpu.async_copy(cidx.at[ci, si, 2 * k + 2], ixa, sxa)
                else:
                    @pl.when(k < nch // 8 - 1)
                    def _():
                        pltpu.make_async_copy(
                            cidx.at[ci, si, 0], ixa, sxa).wait()
                        for t in (0, 1):
                            drain(sss[t])
                            pltpu.async_copy(table.at[ixa.at[t, 0]],
                                             rbufs[t], sgs[t])
                        pltpu.async_copy(cidx.at[ci, si, 2 * k + 3], ixb, sxb)

                    @pl.when(k == nch // 8 - 1)
                    def _():
                        for t in (0, 1):
                            drain(sss[t])

            return carry

        lax.fori_loop(0, nch // 8, super_step, 0)
        plsc.subcore_barrier()
        pltpu.sync_copy(acc.at[pl.ds(si * zr, zr)],
                        out.at[ci, pl.ds(si * zr, zr)])

    return pl.kernel(
        body,
        out_type=jax.ShapeDtypeStruct((2, nd_pad, cw), jnp.float32),
        mesh=mesh,
        compiler_params=pltpu.CompilerParams(use_tc_tiling_on_sc=False),
        scratch_types=[
            pltpu.VMEM((4, 2, _B), jnp.int32),
            pltpu.VMEM((4, 2, _B), jnp.int32),
            pltpu.VMEM((_B, cw), jnp.float32),
            pltpu.VMEM((_B, cw), jnp.float32),
            pltpu.VMEM_SHARED((nd_pad, cw), jnp.float32),
            pltpu.SemaphoreType.DMA,
            pltpu.SemaphoreType.DMA,
            pltpu.SemaphoreType.DMA,
            pltpu.SemaphoreType.DMA,
            pltpu.SemaphoreType.DMA,
            pltpu.SemaphoreType.DMA,
        ],
    )


def _run_pass(table, cidx, nd_pad):
    tn, cw = table.shape
    nch = cidx.shape[2]
    cidx = cidx.reshape(2, _NS, nch // 4, 4, 2, _B)
    zeros = jnp.zeros((nd_pad // _NS, cw), jnp.float32)
    return _sc_pass(tn, cw, nch, nd_pad)(table, cidx, zeros)


_R = 1000
_R2 = 2528


def _ka_body(x_ref, th_ref, b_ref, av_ref, xt_ref, sv_ref, m_ref):
    xt = jnp.dot(x_ref[...], th_ref[...],
                 preferred_element_type=jnp.float32) + b_ref[...]
    xt_ref[...] = xt
    sv = _leaky(jnp.sum((xt * av_ref[...]).reshape(_R, N_HEADS, D_HID), 2))
    sv_ref[...] = sv

    @pl.when(pl.program_id(0) == 0)
    def _():
        m_ref[...] = jnp.full((1, N_HEADS), -jnp.inf, jnp.float32)

    m_ref[...] = jnp.maximum(m_ref[...], jnp.max(sv, 0, keepdims=True))


def _dense1(X, Th, b, av):
    return pl.pallas_call(
        _ka_body,
        grid=(N_V // _R,),
        in_specs=[
            pl.BlockSpec((_R, D_IN), lambda i: (i, 0)),
            pl.BlockSpec((D_IN, N_HEADS * D_HID), lambda i: (0, 0)),
            pl.BlockSpec((1, N_HEADS * D_HID), lambda i: (0, 0)),
            pl.BlockSpec((1, N_HEADS * D_HID), lambda i: (0, 0)),
        ],
        out_specs=[
            pl.BlockSpec((_R, N_HEADS * D_HID), lambda i: (i, 0)),
            pl.BlockSpec((_R, N_HEADS), lambda i: (i, 0)),
            pl.BlockSpec((1, N_HEADS), lambda i: (0, 0)),
        ],
        out_shape=[
            jax.ShapeDtypeStruct((N_V, N_HEADS * D_HID), jnp.float32),
            jax.ShapeDtypeStruct((N_V, N_HEADS), jnp.float32),
            jax.ShapeDtypeStruct((1, N_HEADS), jnp.float32),
        ],
    )(X, Th, b, av)


def _kb_body(xt_ref, sv_ref, m_ref, g1_ref):
    xt = xt_ref[...]
    p = jnp.exp(sv_ref[...] - m_ref[...])
    ones = jnp.ones((_R, 1), jnp.float32)
    pad = jnp.zeros((_R, 13), jnp.float32)
    for c in range(2):
        g1_ref[c] = jnp.concatenate([
            xt[:, 128 * c:128 * c + 64] * p[:, 2 * c:2 * c + 1],
            xt[:, 128 * c + 64:128 * c + 128] * p[:, 2 * c + 1:2 * c + 2],
            p[:, 2 * c:2 * c + 1], p[:, 2 * c + 1:2 * c + 2], ones, pad,
        ], 1)


def _build_g1(Xt, sv, M):
    return pl.pallas_call(
        _kb_body,
        grid=(N_V // _R,),
        in_specs=[
            pl.BlockSpec((_R, N_HEADS * D_HID), lambda i: (i, 0)),
            pl.BlockSpec((_R, N_HEADS), lambda i: (i, 0)),
            pl.BlockSpec((1, N_HEADS), lambda i: (0, 0)),
        ],
        out_specs=pl.BlockSpec((2, _R, C1), lambda i: (0, i, 0)),
        out_shape=jax.ShapeDtypeStruct((2, N_V, C1), jnp.float32),
    )(Xt, sv, M)


def _guard_div(num, den):
    return jnp.where(den > 0, num / jnp.maximum(den, 1e-30), 0.0)


def _kc_body(a_ref, ae_ref, g2_ref):
    ones = jnp.ones((N_E, 1), jnp.float32)
    pad = jnp.zeros((N_E, 13), jnp.float32)
    for c in range(2):
        blk = a_ref[c]
        parts = []
        qs = []
        for h in range(2):
            num = blk[:N_E, 64 * h:64 * (h + 1)]
            den = blk[:N_E, 128 + h:129 + h]
            ye = _guard_div(num, den)
            ae = ae_ref[2 * c + h:2 * c + h + 1, :]
            se = _leaky(jnp.sum(ye * ae, 1, keepdims=True))
            q = jnp.exp(se - jnp.max(se, 0, keepdims=True))
            parts.append(ye * q)
            qs.append(q)
        g2_ref[c] = jnp.concatenate(parts + qs + [ones, pad], 1)


def _edge_stage1(A, ae1):
    return pl.pallas_call(
        _kc_body,
        out_shape=jax.ShapeDtypeStruct((2, N_E, C1), jnp.float32),
    )(A, ae1)


def _ke_body(b_ref, pe_ref, xct_ref):
    parts = []
    for c in range(2):
        for h in range(2):
            num = b_ref[c][:, 64 * h:64 * (h + 1)]
            den = b_ref[c][:, 128 + h:129 + h]
            xo = _guard_div(num, den)
            parts.append(jnp.maximum(xo, 0.0)
                         + jnp.exp(jnp.minimum(xo, 0.0)) - 1.0)
    xc = jnp.concatenate(parts, 1)
    plap = jnp.sum(jnp.abs(xc), 1, keepdims=True)
    xc = xc + plap * pe_ref[...]
    xct_ref[0] = xc[:, :128]
    xct_ref[1] = xc[:, 128:]


def _vertex_stage1(B, pe):
    return pl.pallas_call(
        _ke_body,
        grid=(NV_PAD // _R2,),
        in_specs=[
            pl.BlockSpec((2, _R2, C1), lambda i: (0, i, 0)),
            pl.BlockSpec((_R2, 256), lambda i: (i, 0)),
        ],
        out_specs=pl.BlockSpec((2, _R2, C2), lambda i: (0, i, 0)),
        out_shape=jax.ShapeDtypeStruct((2, NV_PAD, C2), jnp.float32),
    )(B, pe)


def _kf_body(p3_ref, a_ref, ym_ref):
    de = jnp.maximum(a_ref[0][:N_E, 130:131], 1.0)
    for c in range(2):
        ym_ref[c] = p3_ref[c][:N_E, :] / de


def _edge_hspd(P3, A):
    return pl.pallas_call(
        _kf_body,
        out_shape=jax.ShapeDtypeStruct((2, N_E, C2), jnp.float32),
    )(P3, A)


def _kg_body(p4_ref, b_ref, xct_ref, th_ref, b2_ref, av2_ref, d3_ref, m_ref):
    dv = jnp.maximum(b_ref[0][:, 130:131], 1.0)
    xc2 = jnp.concatenate([xct_ref[0] + p4_ref[0] / dv,
                           xct_ref[1] + p4_ref[1] / dv], 1)
    xt2 = jnp.dot(xc2, th_ref[...],
                  preferred_element_type=jnp.float32) + b2_ref[...]
    sv2 = _leaky(jnp.sum(xt2 * av2_ref[...], 1, keepdims=True))
    d3 = jnp.concatenate([xt2, sv2, jnp.zeros((_R2, 7), jnp.float32)], 1)
    d3_ref[...] = d3
    i = pl.program_id(0)
    rows = lax.broadcasted_iota(jnp.int32, (_R2, 1), 0) + i * _R2
    d3m = jnp.where(rows < N_V, d3, -jnp.inf)

    @pl.when(i == 0)
    def _():
        m_ref[...] = jnp.full((1, C3), -jnp.inf, jnp.float32)

    m_ref[...] = jnp.maximum(m_ref[...], jnp.max(d3m, 0, keepdims=True))


def _vertex_stage2(P4, B, XcT, Th2, b2, av2):
    return pl.pallas_call(
        _kg_body,
        grid=(NV_PAD // _R2,),
        in_specs=[
            pl.BlockSpec((2, _R2, C2), lambda i: (0, i, 0)),
            pl.BlockSpec((2, _R2, C1), lambda i: (0, i, 0)),
            pl.BlockSpec((2, _R2, C2), lambda i: (0, i, 0)),
            pl.BlockSpec((256, N_CLASSES), lambda i: (0, 0)),
            pl.BlockSpec((1, N_CLASSES), lambda i: (0, 0)),
            pl.BlockSpec((1, N_CLASSES), lambda i: (0, 0)),
        ],
        out_specs=[
            pl.BlockSpec((_R2, C3), lambda i: (i, 0)),
            pl.BlockSpec((1, C3), lambda i: (0, 0)),
        ],
        out_shape=[
            jax.ShapeDtypeStruct((NV_PAD, C3), jnp.float32),
            jax.ShapeDtypeStruct((1, C3), jnp.float32),
        ],
    )(P4, B, XcT, Th2, b2, av2)


def _kh_body(d3_ref, m_ref, g3_ref):
    d3 = d3_ref[...]
    p2 = jnp.exp(d3[:, 40:41] - m_ref[0:1, 40:41])
    g3_ref[...] = jnp.concatenate(
        [d3[:, :40] * p2, p2, jnp.zeros((NV_PAD, 7), jnp.float32)], 1)


def _build_g3(D3, M3):
    return pl.pallas_call(
        _kh_body,
        out_shape=jax.ShapeDtypeStruct((NV_PAD, C3), jnp.float32),
    )(D3, M3)


def _ki_body(p5_ref, ae2_ref, g4_ref):
    a2 = p5_ref[0][:N_E, :] + p5_ref[1][:N_E, :]
    ye2 = _guard_div(a2[:, :40], a2[:, 40:41])
    se2 = _leaky(jnp.sum(ye2 * ae2_ref[...], 1, keepdims=True))
    q2 = jnp.exp(se2 - jnp.max(se2, 0, keepdims=True))
    g4_ref[...] = jnp.concatenate(
        [ye2 * q2, q2, jnp.zeros((N_E, 7), jnp.float32)], 1)


def _edge_stage2(P5, ae2):
    return pl.pallas_call(
        _ki_body,
        out_shape=jax.ShapeDtypeStruct((N_E, C3), jnp.float32),
    )(P5, ae2)


def _kj_body(p6_ref, out_ref):
    bs = p6_ref[0][:N_V, :] + p6_ref[1][:N_V, :]
    out_ref[...] = _guard_div(bs[:, :40], bs[:, 40:41])


def _final(P6):
    return pl.pallas_call(
        _kj_body,
        out_shape=jax.ShapeDtypeStruct((N_V, N_CLASSES), jnp.float32),
    )(P6)


def _pos_encoding_const(n, channels):
    positions = jnp.arange(n, dtype=jnp.float32)[:, None]
    div1 = jnp.exp(jnp.arange(0, channels, 2, dtype=jnp.float32)
                   * (-math.log(10000.0) / channels))
    div2 = jnp.exp(jnp.arange(1, channels, 2, dtype=jnp.float32)
                   * (-math.log(10000.0) / channels))
    pe = jnp.zeros((n, channels), dtype=jnp.float32)
    pe = pe.at[:, 0::2].set(jnp.sin(positions * div1) / channels ** 0.5)
    pe = pe.at[:, 1::2].set(jnp.cos(positions * div2) / channels ** 0.5)
    return pe


def _pad_reshape(idx2, fill, kt):
    n = idx2.shape[1]
    total = 16 * kt
    pad = jnp.full((2, total - n), fill, jnp.int32)
    return jnp.concatenate([idx2, pad], 1).reshape(2, _NS, kt // _B, _B)


KT1 = 20480
KT2 = 10240


def kernel(X, v_idx, e_idx, Theta1, b1, av1, ae1, Theta2, b2, av2, ae2):
    v2 = jnp.stack([v_idx, v_idx])
    e2 = jnp.stack([e_idx, e_idx])
    off_v = jnp.array([[0], [N_V]], jnp.int32)
    off_vp = jnp.array([[0], [NV_PAD]], jnp.int32)
    off_e = jnp.array([[0], [N_E]], jnp.int32)
    comb = lambda s, d: jnp.stack([s, d], 3)
    d_e = _pad_reshape(e2, NE_PAD - 1, KT1)
    d_v = _pad_reshape(v2, NV_PAD - 1, KT1)
    x_p1 = comb(_pad_reshape(v2 + off_v, 0, KT1), d_e)
    x_p2 = comb(_pad_reshape(e2 + off_e, 0, KT1), d_v)
    x_p3 = comb(_pad_reshape(v2 + off_vp, 0, KT1), d_e)
    x_p4 = comb(_pad_reshape(e2 + off_e, 0, KT1), d_v)
    x_p5 = comb(_pad_reshape(v_idx.reshape(2, -1), 0, KT2),
                _pad_reshape(e_idx.reshape(2, -1), NE_PAD - 1, KT2))
    x_p6 = comb(_pad_reshape(e_idx.reshape(2, -1), 0, KT2),
                _pad_reshape(v_idx.reshape(2, -1), NV_PAD - 1, KT2))
    pe = jnp.concatenate([_pos_encoding_const(N_V, 256),
                          jnp.zeros((NV_PAD - N_V, 256), jnp.float32)])

    Th = jnp.transpose(Theta1, (1, 0, 2)).reshape(D_IN, N_HEADS * D_HID)
    Xt, sv, M = _dense1(X, Th, b1.reshape(1, -1), av1.reshape(1, -1))
    G1 = _build_g1(Xt, sv, M)

    A = _run_pass(G1.reshape(2 * N_V, C1), x_p1, NE_PAD)
    G2 = _edge_stage1(A, ae1)

    B = _run_pass(G2.reshape(2 * N_E, C1), x_p2, NV_PAD)
    XcT = _vertex_stage1(B, pe)

    P3 = _run_pass(XcT.reshape(2 * NV_PAD, C2), x_p3, NE_PAD)
    Ym = _edge_hspd(P3, A)
    P4 = _run_pass(Ym.reshape(2 * N_E, C2), x_p4, NV_PAD)

    D3, M3 = _vertex_stage2(P4, B, XcT, Theta2,
                            b2.reshape(1, -1), av2.reshape(1, -1))
    G3 = _build_g3(D3, M3)

    P5 = _run_pass(G3, x_p5, NE_PAD)
    G4 = _edge_stage2(P5, ae2.reshape(1, -1))
    P6 = _run_pass(G4, x_p6, NV_PAD)
    return _final(P6)


# speedup vs baseline: 1.5019x; 1.5019x over previous
import functools
import math

import jax
import jax.numpy as jnp
from jax import lax
from jax.experimental import pallas as pl
from jax.experimental.pallas import tpu as pltpu
from jax.experimental.pallas import tpu_sc as plsc

N_V = 10000
N_E = 2500
NNZ = 320000
D_IN = 128
D_HID = 64
N_HEADS = 4
N_CLASSES = 40
NEG_SLOPE = 0.2

NV_PAD = 10112
NE_PAD = 2560
_NS = 16
_B = 128
C1 = 144
C2 = 128
C3 = 48


def _leaky(x):
    return jnp.where(x >= 0, x, NEG_SLOPE * x)


@functools.lru_cache(maxsize=None)
def _sc_pass(tn, cw, nch, nd_pad):
    zr = nd_pad // _NS
    mesh = plsc.VectorSubcoreMesh(core_axis_name="c", subcore_axis_name="s",
                                  num_cores=2, num_subcores=_NS)

    def body(table, cidx, zeros, out, ixa, ixb, r0, r1, acc,
             sg0, sg1, ss0, ss1, sxa, sxb):
        ci = lax.axis_index("c")
        si = lax.axis_index("s")
        pltpu.sync_copy(zeros, acc.at[pl.ds(si * zr, zr)])
        pltpu.sync_copy(cidx.at[ci, si, 0], ixa)
        pltpu.sync_copy(cidx.at[ci, si, 1], ixb)
        plsc.subcore_barrier()

        rbufs = (r0, r1)
        sgs = (sg0, sg1)
        sss = (ss0, ss1)

        def drain(sem):
            pltpu.make_async_copy(table.at[pl.ds(0, _B)], r0, sem).wait()

        pltpu.async_copy(table.at[ixa.at[0, 0]], r0, sg0)
        pltpu.async_copy(table.at[ixa.at[1, 0]], r1, sg1)

        def super_step(k, carry):
            for half, (ixc, ixn, sxc) in enumerate(
                    ((ixa, ixb, sxa), (ixb, ixa, sxb))):
                for t in (0, 1):
                    drain(sgs[t])
                    pltpu.async_copy(rbufs[t], acc.at[ixc.at[t, 1]],
                                     sss[t], add=True)
                for t in (0, 1):
                    drain(sss[t])
                    pltpu.async_copy(table.at[ixc.at[2 + t, 0]],
                                     rbufs[t], sgs[t])
                for t in (0, 1):
                    drain(sgs[t])
                    pltpu.async_copy(rbufs[t], acc.at[ixc.at[2 + t, 1]],
                                     sss[t], add=True)
                if half == 0:
                    @pl.when(k > 0)
                    def _():
                        pltpu.make_async_copy(
                            cidx.at[ci, si, 0], ixb, sxb).wait()

                    for t in (0, 1):
                        drain(sss[t])
                        pltpu.async_copy(table.at[ixn.at[t, 0]],
                                         rbufs[t], sgs[t])

                    @pl.when(k < nch // 8 - 1)
                    def _():
                        pltpu.async_copy(cidx.at[ci, si, 2 * k + 2], ixa, sxa)
                else:
                    @pl.when(k < nch // 8 - 1)
                    def _():
                        pltpu.make_async_copy(
                            cidx.at[ci, si, 0], ixa, sxa).wait()
                        for t in (0, 1):
                            drain(sss[t])
                            pltpu.async_copy(table.at[ixa.at[t, 0]],
                                             rbufs[t], sgs[t])
                        pltpu.async_copy(cidx.at[ci, si, 2 * k + 3], ixb, sxb)

                    @pl.when(k == nch // 8 - 1)
                    def _():
                        for t in (0, 1):
                            drain(sss[t])

            return carry

        lax.fori_loop(0, nch // 8, super_step, 0)
        plsc.subcore_barrier()
        pltpu.sync_copy(acc.at[pl.ds(si * zr, zr)],
                        out.at[ci, pl.ds(si * zr, zr)])

    return pl.kernel(
        body,
        out_type=jax.ShapeDtypeStruct((2, nd_pad, cw), jnp.float32),
        mesh=mesh,
        compiler_params=pltpu.CompilerParams(use_tc_tiling_on_sc=False),
        scratch_types=[
            pltpu.VMEM((4, 2, _B), jnp.int32),
            pltpu.VMEM((4, 2, _B), jnp.int32),
            pltpu.VMEM((_B, cw), jnp.float32),
            pltpu.VMEM((_B, cw), jnp.float32),
            pltpu.VMEM_SHARED((nd_pad, cw), jnp.float32),
            pltpu.SemaphoreType.DMA,
            pltpu.SemaphoreType.DMA,
            pltpu.SemaphoreType.DMA,
            pltpu.SemaphoreType.DMA,
            pltpu.SemaphoreType.DMA,
            pltpu.SemaphoreType.DMA,
        ],
    )


def _run_pass(table, cidx, nd_pad):
    tn, cw = table.shape
    nch = cidx.shape[2]
    cidx = cidx.reshape(2, _NS, nch // 4, 4, 2, _B)
    zeros = jnp.zeros((nd_pad // _NS, cw), jnp.float32)
    return _sc_pass(tn, cw, nch, nd_pad)(table, cidx, zeros)


_R = 1000
_R2 = 2528


def _ka_body(x_ref, th_ref, b_ref, av_ref, xt_ref, sv_ref, m_ref):
    xt = jnp.dot(x_ref[...], th_ref[...],
                 preferred_element_type=jnp.float32) + b_ref[...]
    xt_ref[...] = xt
    sv = _leaky(jnp.sum((xt * av_ref[...]).reshape(_R, N_HEADS, D_HID), 2))
    sv_ref[...] = sv

    @pl.when(pl.program_id(0) == 0)
    def _():
        m_ref[...] = jnp.full((1, N_HEADS), -jnp.inf, jnp.float32)

    m_ref[...] = jnp.maximum(m_ref[...], jnp.max(sv, 0, keepdims=True))


def _dense1(X, Th, b, av):
    return pl.pallas_call(
        _ka_body,
        grid=(N_V // _R,),
        in_specs=[
            pl.BlockSpec((_R, D_IN), lambda i: (i, 0)),
            pl.BlockSpec((D_IN, N_HEADS * D_HID), lambda i: (0, 0)),
            pl.BlockSpec((1, N_HEADS * D_HID), lambda i: (0, 0)),
            pl.BlockSpec((1, N_HEADS * D_HID), lambda i: (0, 0)),
        ],
        out_specs=[
            pl.BlockSpec((_R, N_HEADS * D_HID), lambda i: (i, 0)),
            pl.BlockSpec((_R, N_HEADS), lambda i: (i, 0)),
            pl.BlockSpec((1, N_HEADS), lambda i: (0, 0)),
        ],
        out_shape=[
            jax.ShapeDtypeStruct((N_V, N_HEADS * D_HID), jnp.float32),
            jax.ShapeDtypeStruct((N_V, N_HEADS), jnp.float32),
            jax.ShapeDtypeStruct((1, N_HEADS), jnp.float32),
        ],
    )(X, Th, b, av)


def _kb_body(xt_ref, sv_ref, m_ref, g1_ref):
    xt = xt_ref[...]
    p = jnp.exp(sv_ref[...] - m_ref[...])
    ones = jnp.ones((_R, 1), jnp.float32)
    pad = jnp.zeros((_R, 13), jnp.float32)
    for c in range(2):
        g1_ref[c] = jnp.concatenate([
            xt[:, 128 * c:128 * c + 64] * p[:, 2 * c:2 * c + 1],
            xt[:, 128 * c + 64:128 * c + 128] * p[:, 2 * c + 1:2 * c + 2],
            p[:, 2 * c:2 * c + 1], p[:, 2 * c + 1:2 * c + 2], ones, pad,
        ], 1)


def _build_g1(Xt, sv, M):
    return pl.pallas_call(
        _kb_body,
        grid=(N_V // _R,),
        in_specs=[
            pl.BlockSpec((_R, N_HEADS * D_HID), lambda i: (i, 0)),
            pl.BlockSpec((_R, N_HEADS), lambda i: (i, 0)),
            pl.BlockSpec((1, N_HEADS), lambda i: (0, 0)),
        ],
        out_specs=pl.BlockSpec((2, _R, C1), lambda i: (0, i, 0)),
        out_shape=jax.ShapeDtypeStruct((2, N_V, C1), jnp.float32),
    )(Xt, sv, M)


def _guard_div(num, den):
    return jnp.where(den > 0, num / jnp.maximum(den, 1e-30), 0.0)


def _kc_body(a_ref, ae_ref, g2_ref):
    ones = jnp.ones((N_E, 1), jnp.float32)
    pad = jnp.zeros((N_E, 13), jnp.float32)
    for c in range(2):
        blk = a_ref[c]
        parts = []
        qs = []
        for h in range(2):
            num = blk[:N_E, 64 * h:64 * (h + 1)]
            den = blk[:N_E, 128 + h:129 + h]
            ye = _guard_div(num, den)
            ae = ae_ref[2 * c + h:2 * c + h + 1, :]
            se = _leaky(jnp.sum(ye * ae, 1, keepdims=True))
            q = jnp.exp(se - jnp.max(se, 0, keepdims=True))
            parts.append(ye * q)
            qs.append(q)
        g2_ref[c] = jnp.concatenate(parts + qs + [ones, pad], 1)


def _edge_stage1(A, ae1):
    return pl.pallas_call(
        _kc_body,
        out_shape=jax.ShapeDtypeStruct((2, N_E, C1), jnp.float32),
    )(A, ae1)


def _ke_body(b_ref, pe_ref, th_ref, u_ref):
    parts = []
    for c in range(2):
        for h in range(2):
            num = b_ref[c][:, 64 * h:64 * (h + 1)]
            den = b_ref[c][:, 128 + h:129 + h]
            xo = _guard_div(num, den)
            parts.append(jnp.maximum(xo, 0.0)
                         + jnp.exp(jnp.minimum(xo, 0.0)) - 1.0)
    xc = jnp.concatenate(parts, 1)
    plap = jnp.sum(jnp.abs(xc), 1, keepdims=True)
    xc = xc + plap * pe_ref[...]
    u = jnp.dot(xc, th_ref[...], preferred_element_type=jnp.float32)
    u_ref[...] = jnp.concatenate([u, jnp.zeros((_R2, 8), jnp.float32)], 1)


def _vertex_stage1(B, pe, Th2):
    return pl.pallas_call(
        _ke_body,
        grid=(NV_PAD // _R2,),
        in_specs=[
            pl.BlockSpec((2, _R2, C1), lambda i: (0, i, 0)),
            pl.BlockSpec((_R2, 256), lambda i: (i, 0)),
            pl.BlockSpec((256, N_CLASSES), lambda i: (0, 0)),
        ],
        out_specs=pl.BlockSpec((_R2, C3), lambda i: (i, 0)),
        out_shape=jax.ShapeDtypeStruct((NV_PAD, C3), jnp.float32),
    )(B, pe, Th2)


def _kf_body(p3_ref, a_ref, ym_ref):
    de = jnp.maximum(a_ref[0][:N_E, 130:131], 1.0)
    s = p3_ref[0][:N_E, :] + p3_ref[1][:N_E, :]
    ym_ref[...] = s / de


def _edge_hspd(P3, A):
    return pl.pallas_call(
        _kf_body,
        out_shape=jax.ShapeDtypeStruct((N_E, C3), jnp.float32),
    )(P3, A)


def _kg_body(p4_ref, b_ref, u_ref, b2_ref, av2_ref, d3_ref, m_ref):
    dv = jnp.maximum(b_ref[0][:, 130:131], 1.0)
    hs = (p4_ref[0][:, :40] + p4_ref[1][:, :40]) / dv
    xt2 = u_ref[...][:, :40] + hs + b2_ref[...]
    sv2 = _leaky(jnp.sum(xt2 * av2_ref[...], 1, keepdims=True))
    d3 = jnp.concatenate([xt2, sv2, jnp.zeros((_R2, 7), jnp.float32)], 1)
    d3_ref[...] = d3
    i = pl.program_id(0)
    rows = lax.broadcasted_iota(jnp.int32, (_R2, 1), 0) + i * _R2
    d3m = jnp.where(rows < N_V, d3, -jnp.inf)

    @pl.when(i == 0)
    def _():
        m_ref[...] = jnp.full((1, C3), -jnp.inf, jnp.float32)

    m_ref[...] = jnp.maximum(m_ref[...], jnp.max(d3m, 0, keepdims=True))


def _vertex_stage2(P4, B, U, b2, av2):
    return pl.pallas_call(
        _kg_body,
        grid=(NV_PAD // _R2,),
        in_specs=[
            pl.BlockSpec((2, _R2, C3), lambda i: (0, i, 0)),
            pl.BlockSpec((2, _R2, C1), lambda i: (0, i, 0)),
            pl.BlockSpec((_R2, C3), lambda i: (i, 0)),
            pl.BlockSpec((1, N_CLASSES), lambda i: (0, 0)),
            pl.BlockSpec((1, N_CLASSES), lambda i: (0, 0)),
        ],
        out_specs=[
            pl.BlockSpec((_R2, C3), lambda i: (i, 0)),
            pl.BlockSpec((1, C3), lambda i: (0, 0)),
        ],
        out_shape=[
            jax.ShapeDtypeStruct((NV_PAD, C3), jnp.float32),
            jax.ShapeDtypeStruct((1, C3), jnp.float32),
        ],
    )(P4, B, U, b2, av2)


def _kh_body(d3_ref, m_ref, g3_ref):
    d3 = d3_ref[...]
    p2 = jnp.exp(d3[:, 40:41] - m_ref[0:1, 40:41])
    g3_ref[...] = jnp.concatenate(
        [d3[:, :40] * p2, p2, jnp.zeros((NV_PAD, 7), jnp.float32)], 1)


def _build_g3(D3, M3):
    return pl.pallas_call(
        _kh_body,
        out_shape=jax.ShapeDtypeStruct((NV_PAD, C3), jnp.float32),
    )(D3, M3)


def _ki_body(p5_ref, ae2_ref, g4_ref):
    a2 = p5_ref[0][:N_E, :] + p5_ref[1][:N_E, :]
    ye2 = _guard_div(a2[:, :40], a2[:, 40:41])
    se2 = _leaky(jnp.sum(ye2 * ae2_ref[...], 1, keepdims=True))
    q2 = jnp.exp(se2 - jnp.max(se2, 0, keepdims=True))
    g4_ref[...] = jnp.concatenate(
        [ye2 * q2, q2, jnp.zeros((N_E, 7), jnp.float32)], 1)


def _edge_stage2(P5, ae2):
    return pl.pallas_call(
        _ki_body,
        out_shape=jax.ShapeDtypeStruct((N_E, C3), jnp.float32),
    )(P5, ae2)


def _kj_body(p6_ref, out_ref):
    bs = p6_ref[0][:N_V, :] + p6_ref[1][:N_V, :]
    out_ref[...] = _guard_div(bs[:, :40], bs[:, 40:41])


def _final(P6):
    return pl.pallas_call(
        _kj_body,
        out_shape=jax.ShapeDtypeStruct((N_V, N_CLASSES), jnp.float32),
    )(P6)


def _pos_encoding_const(n, channels):
    positions = jnp.arange(n, dtype=jnp.float32)[:, None]
    div1 = jnp.exp(jnp.arange(0, channels, 2, dtype=jnp.float32)
                   * (-math.log(10000.0) / channels))
    div2 = jnp.exp(jnp.arange(1, channels, 2, dtype=jnp.float32)
                   * (-math.log(10000.0) / channels))
    pe = jnp.zeros((n, channels), dtype=jnp.float32)
    pe = pe.at[:, 0::2].set(jnp.sin(positions * div1) / channels ** 0.5)
    pe = pe.at[:, 1::2].set(jnp.cos(positions * div2) / channels ** 0.5)
    return pe


def _pad_reshape(idx2, fill, kt):
    n = idx2.shape[1]
    total = 16 * kt
    pad = jnp.full((2, total - n), fill, jnp.int32)
    return jnp.concatenate([idx2, pad], 1).reshape(2, _NS, kt // _B, _B)


KT1 = 20480
KT2 = 10240


def kernel(X, v_idx, e_idx, Theta1, b1, av1, ae1, Theta2, b2, av2, ae2):
    v2 = jnp.stack([v_idx, v_idx])
    e2 = jnp.stack([e_idx, e_idx])
    off_v = jnp.array([[0], [N_V]], jnp.int32)
    off_vp = jnp.array([[0], [NV_PAD]], jnp.int32)
    off_e = jnp.array([[0], [N_E]], jnp.int32)
    comb = lambda s, d: jnp.stack([s, d], 3)
    d_e = _pad_reshape(e2, NE_PAD - 1, KT1)
    d_v = _pad_reshape(v2, NV_PAD - 1, KT1)
    x_p1 = comb(_pad_reshape(v2 + off_v, 0, KT1), d_e)
    x_p2 = comb(_pad_reshape(e2 + off_e, 0, KT1), d_v)
    x_p5 = comb(_pad_reshape(v_idx.reshape(2, -1), 0, KT2),
                _pad_reshape(e_idx.reshape(2, -1), NE_PAD - 1, KT2))
    x_p6 = comb(_pad_reshape(e_idx.reshape(2, -1), 0, KT2),
                _pad_reshape(v_idx.reshape(2, -1), NV_PAD - 1, KT2))
    pe = jnp.concatenate([_pos_encoding_const(N_V, 256),
                          jnp.zeros((NV_PAD - N_V, 256), jnp.float32)])

    Th = jnp.transpose(Theta1, (1, 0, 2)).reshape(D_IN, N_HEADS * D_HID)
    Xt, sv, M = _dense1(X, Th, b1.reshape(1, -1), av1.reshape(1, -1))
    G1 = _build_g1(Xt, sv, M)

    A = _run_pass(G1.reshape(2 * N_V, C1), x_p1, NE_PAD)
    G2 = _edge_stage1(A, ae1)

    B = _run_pass(G2.reshape(2 * N_E, C1), x_p2, NV_PAD)
    U = _vertex_stage1(B, pe, Theta2)

    P3 = _run_pass(U, x_p5, NE_PAD)
    Ym = _edge_hspd(P3, A)
    P4 = _run_pass(Ym, x_p6, NV_PAD)

    D3, M3 = _vertex_stage2(P4, B, U,
                            b2.reshape(1, -1), av2.reshape(1, -1))
    G3 = _build_g3(D3, M3)

    P5 = _run_pass(G3, x_p5, NE_PAD)
    G4 = _edge_stage2(P5, ae2.reshape(1, -1))
    P6 = _run_pass(G4, x_p6, NV_PAD)
    return _final(P6)


# speedup vs baseline: 1.7222x; 1.1467x over previous
import functools
import math

import jax
import jax.numpy as jnp
from jax import lax
from jax.experimental import pallas as pl
from jax.experimental.pallas import tpu as pltpu
from jax.experimental.pallas import tpu_sc as plsc

N_V = 10000
N_E = 2500
NNZ = 320000
D_IN = 128
D_HID = 64
N_HEADS = 4
N_CLASSES = 40
NEG_SLOPE = 0.2

NV_PAD = 10112
NE_PAD = 2560
_NS = 16
_B = 128
C1 = 144
C2 = 128
C3 = 48


def _leaky(x):
    return jnp.where(x >= 0, x, NEG_SLOPE * x)


@functools.lru_cache(maxsize=None)
def _sc_pass(tn, cw, nch, nd_pad):
    zr = nd_pad // _NS
    mesh = plsc.VectorSubcoreMesh(core_axis_name="c", subcore_axis_name="s",
                                  num_cores=2, num_subcores=_NS)

    def body(table, cidx, zeros, out, ixa, ixb, r0, r1, acc,
             sg0, sg1, ss0, ss1, sxa, sxb):
        ci = lax.axis_index("c")
        si = lax.axis_index("s")
        pltpu.sync_copy(zeros, acc.at[pl.ds(si * zr, zr)])
        pltpu.sync_copy(cidx.at[ci, si, 0], ixa)
        pltpu.sync_copy(cidx.at[ci, si, 1], ixb)
        plsc.subcore_barrier()

        rbufs = (r0, r1)
        sgs = (sg0, sg1)
        sss = (ss0, ss1)

        def drain(sem):
            pltpu.make_async_copy(table.at[pl.ds(0, _B)], r0, sem).wait()

        pltpu.async_copy(table.at[ixa.at[0, 0]], r0, sg0)
        pltpu.async_copy(table.at[ixa.at[1, 0]], r1, sg1)

        def super_step(k, carry):
            for half, (ixc, ixn, sxc) in enumerate(
                    ((ixa, ixb, sxa), (ixb, ixa, sxb))):
                for t in (0, 1):
                    drain(sgs[t])
                    pltpu.async_copy(rbufs[t], acc.at[ixc.at[t, 1]],
                                     sss[t], add=True)
                for t in (0, 1):
                    drain(sss[t])
                    pltpu.async_copy(table.at[ixc.at[2 + t, 0]],
                                     rbufs[t], sgs[t])
                for t in (0, 1):
                    drain(sgs[t])
                    pltpu.async_copy(rbufs[t], acc.at[ixc.at[2 + t, 1]],
                                     sss[t], add=True)
                if half == 0:
                    @pl.when(k > 0)
                    def _():
                        pltpu.make_async_copy(
                            cidx.at[ci, si, 0], ixb, sxb).wait()

                    for t in (0, 1):
                        drain(sss[t])
                        pltpu.async_copy(table.at[ixn.at[t, 0]],
                                         rbufs[t], sgs[t])

                    @pl.when(k < nch // 8 - 1)
                    def _():
                        pltpu.async_copy(cidx.at[ci, si, 2 * k + 2], ixa, sxa)
                else:
                    @pl.when(k < nch // 8 - 1)
                    def _():
                        pltpu.make_async_copy(
                            cidx.at[ci, si, 0], ixa, sxa).wait()
                        for t in (0, 1):
                            drain(sss[t])
                            pltpu.async_copy(table.at[ixa.at[t, 0]],
                                             rbufs[t], sgs[t])
                        pltpu.async_copy(cidx.at[ci, si, 2 * k + 3], ixb, sxb)

                    @pl.when(k == nch // 8 - 1)
                    def _():
                        for t in (0, 1):
                            drain(sss[t])

            return carry

        lax.fori_loop(0, nch // 8, super_step, 0)
        plsc.subcore_barrier()
        pltpu.sync_copy(acc.at[pl.ds(si * zr, zr)],
                        out.at[ci, pl.ds(si * zr, zr)])

    return pl.kernel(
        body,
        out_type=jax.ShapeDtypeStruct((2, nd_pad, cw), jnp.float32),
        mesh=mesh,
        compiler_params=pltpu.CompilerParams(use_tc_tiling_on_sc=False),
        scratch_types=[
            pltpu.VMEM((4, 2, _B), jnp.int32),
            pltpu.VMEM((4, 2, _B), jnp.int32),
            pltpu.VMEM((_B, cw), jnp.float32),
            pltpu.VMEM((_B, cw), jnp.float32),
            pltpu.VMEM_SHARED((nd_pad, cw), jnp.float32),
            pltpu.SemaphoreType.DMA,
            pltpu.SemaphoreType.DMA,
            pltpu.SemaphoreType.DMA,
            pltpu.SemaphoreType.DMA,
            pltpu.SemaphoreType.DMA,
            pltpu.SemaphoreType.DMA,
        ],
    )


def _run_pass(table, cidx, nd_pad):
    tn, cw = table.shape
    nch = cidx.shape[2]
    cidx = cidx.reshape(2, _NS, nch // 4, 4, 2, _B)
    zeros = jnp.zeros((nd_pad // _NS, cw), jnp.float32)
    return _sc_pass(tn, cw, nch, nd_pad)(table, cidx, zeros)


_R = 1000
_R2 = 2528


def _ka_body(x_ref, th_ref, b_ref, av_ref, xt_ref, sv_ref, m_ref):
    xt = jnp.dot(x_ref[...], th_ref[...],
                 preferred_element_type=jnp.float32) + b_ref[...]
    xt_ref[...] = xt
    sv = _leaky(jnp.sum((xt * av_ref[...]).reshape(_R, N_HEADS, D_HID), 2))
    sv_ref[...] = sv

    @pl.when(pl.program_id(0) == 0)
    def _():
        m_ref[...] = jnp.full((1, N_HEADS), -jnp.inf, jnp.float32)

    m_ref[...] = jnp.maximum(m_ref[...], jnp.max(sv, 0, keepdims=True))


def _dense1(X, Th, b, av):
    return pl.pallas_call(
        _ka_body,
        grid=(N_V // _R,),
        in_specs=[
            pl.BlockSpec((_R, D_IN), lambda i: (i, 0)),
            pl.BlockSpec((D_IN, N_HEADS * D_HID), lambda i: (0, 0)),
            pl.BlockSpec((1, N_HEADS * D_HID), lambda i: (0, 0)),
            pl.BlockSpec((1, N_HEADS * D_HID), lambda i: (0, 0)),
        ],
        out_specs=[
            pl.BlockSpec((_R, N_HEADS * D_HID), lambda i: (i, 0)),
            pl.BlockSpec((_R, N_HEADS), lambda i: (i, 0)),
            pl.BlockSpec((1, N_HEADS), lambda i: (0, 0)),
        ],
        out_shape=[
            jax.ShapeDtypeStruct((N_V, N_HEADS * D_HID), jnp.float32),
            jax.ShapeDtypeStruct((N_V, N_HEADS), jnp.float32),
            jax.ShapeDtypeStruct((1, N_HEADS), jnp.float32),
        ],
    )(X, Th, b, av)


C0 = 272


def _kb_body(xt_ref, sv_ref, m_ref, g1_ref):
    xt = xt_ref[...]
    p = jnp.exp(sv_ref[...] - m_ref[...])
    g1_ref[...] = jnp.concatenate(
        [xt[:, 64 * h:64 * h + 64] * p[:, h:h + 1] for h in range(4)]
        + [p, jnp.ones((_R, 1), jnp.float32),
           jnp.zeros((_R, 11), jnp.float32)], 1)


def _build_g1(Xt, sv, M):
    return pl.pallas_call(
        _kb_body,
        grid=(N_V // _R,),
        in_specs=[
            pl.BlockSpec((_R, N_HEADS * D_HID), lambda i: (i, 0)),
            pl.BlockSpec((_R, N_HEADS), lambda i: (i, 0)),
            pl.BlockSpec((1, N_HEADS), lambda i: (0, 0)),
        ],
        out_specs=pl.BlockSpec((_R, C0), lambda i: (i, 0)),
        out_shape=jax.ShapeDtypeStruct((N_V, C0), jnp.float32),
    )(Xt, sv, M)


def _guard_div(num, den):
    return jnp.where(den > 0, num / jnp.maximum(den, 1e-30), 0.0)


def _kc_body(a_ref, ae_ref, g2_ref):
    ones = jnp.ones((N_E, 1), jnp.float32)
    pad = jnp.zeros((N_E, 13), jnp.float32)
    a2 = a_ref[0][:N_E, :] + a_ref[1][:N_E, :]
    for c in range(2):
        parts = []
        qs = []
        for h in range(2):
            hh = 2 * c + h
            num = a2[:, 64 * hh:64 * (hh + 1)]
            den = a2[:, 256 + hh:257 + hh]
            ye = _guard_div(num, den)
            ae = ae_ref[hh:hh + 1, :]
            se = _leaky(jnp.sum(ye * ae, 1, keepdims=True))
            q = jnp.exp(se - jnp.max(se, 0, keepdims=True))
            parts.append(ye * q)
            qs.append(q)
        g2_ref[c] = jnp.concatenate(parts + qs + [ones, pad], 1)


def _edge_stage1(A, ae1):
    return pl.pallas_call(
        _kc_body,
        out_shape=jax.ShapeDtypeStruct((2, N_E, C1), jnp.float32),
    )(A, ae1)


def _ke_body(b_ref, pe_ref, th_ref, u_ref):
    parts = []
    for c in range(2):
        for h in range(2):
            num = b_ref[c][:, 64 * h:64 * (h + 1)]
            den = b_ref[c][:, 128 + h:129 + h]
            xo = _guard_div(num, den)
            parts.append(jnp.maximum(xo, 0.0)
                         + jnp.exp(jnp.minimum(xo, 0.0)) - 1.0)
    xc = jnp.concatenate(parts, 1)
    plap = jnp.sum(jnp.abs(xc), 1, keepdims=True)
    xc = xc + plap * pe_ref[...]
    u = jnp.dot(xc, th_ref[...], preferred_element_type=jnp.float32)
    u_ref[...] = jnp.concatenate([u, jnp.zeros((_R2, 8), jnp.float32)], 1)


def _vertex_stage1(B, pe, Th2):
    return pl.pallas_call(
        _ke_body,
        grid=(NV_PAD // _R2,),
        in_specs=[
            pl.BlockSpec((2, _R2, C1), lambda i: (0, i, 0)),
            pl.BlockSpec((_R2, 256), lambda i: (i, 0)),
            pl.BlockSpec((256, N_CLASSES), lambda i: (0, 0)),
        ],
        out_specs=pl.BlockSpec((_R2, C3), lambda i: (i, 0)),
        out_shape=jax.ShapeDtypeStruct((NV_PAD, C3), jnp.float32),
    )(B, pe, Th2)


def _kf_body(p3_ref, a_ref, ym_ref):
    de = jnp.maximum(a_ref[0][:N_E, 260:261] + a_ref[1][:N_E, 260:261], 1.0)
    s = p3_ref[0][:N_E, :] + p3_ref[1][:N_E, :]
    ym_ref[...] = s / de


def _edge_hspd(P3, A):
    return pl.pallas_call(
        _kf_body,
        out_shape=jax.ShapeDtypeStruct((N_E, C3), jnp.float32),
    )(P3, A)


def _kg_body(p4_ref, b_ref, u_ref, b2_ref, av2_ref, d3_ref, m_ref):
    dv = jnp.maximum(b_ref[0][:, 130:131], 1.0)
    hs = (p4_ref[0][:, :40] + p4_ref[1][:, :40]) / dv
    xt2 = u_ref[...][:, :40] + hs + b2_ref[...]
    sv2 = _leaky(jnp.sum(xt2 * av2_ref[...], 1, keepdims=True))
    d3 = jnp.concatenate([xt2, sv2, jnp.zeros((_R2, 7), jnp.float32)], 1)
    d3_ref[...] = d3
    i = pl.program_id(0)
    rows = lax.broadcasted_iota(jnp.int32, (_R2, 1), 0) + i * _R2
    d3m = jnp.where(rows < N_V, d3, -jnp.inf)

    @pl.when(i == 0)
    def _():
        m_ref[...] = jnp.full((1, C3), -jnp.inf, jnp.float32)

    m_ref[...] = jnp.maximum(m_ref[...], jnp.max(d3m, 0, keepdims=True))


def _vertex_stage2(P4, B, U, b2, av2):
    return pl.pallas_call(
        _kg_body,
        grid=(NV_PAD // _R2,),
        in_specs=[
            pl.BlockSpec((2, _R2, C3), lambda i: (0, i, 0)),
            pl.BlockSpec((2, _R2, C1), lambda i: (0, i, 0)),
            pl.BlockSpec((_R2, C3), lambda i: (i, 0)),
            pl.BlockSpec((1, N_CLASSES), lambda i: (0, 0)),
            pl.BlockSpec((1, N_CLASSES), lambda i: (0, 0)),
        ],
        out_specs=[
            pl.BlockSpec((_R2, C3), lambda i: (i, 0)),
            pl.BlockSpec((1, C3), lambda i: (0, 0)),
        ],
        out_shape=[
            jax.ShapeDtypeStruct((NV_PAD, C3), jnp.float32),
            jax.ShapeDtypeStruct((1, C3), jnp.float32),
        ],
    )(P4, B, U, b2, av2)


def _kh_body(d3_ref, m_ref, g3_ref):
    d3 = d3_ref[...]
    p2 = jnp.exp(d3[:, 40:41] - m_ref[0:1, 40:41])
    g3_ref[...] = jnp.concatenate(
        [d3[:, :40] * p2, p2, jnp.zeros((NV_PAD, 7), jnp.float32)], 1)


def _build_g3(D3, M3):
    return pl.pallas_call(
        _kh_body,
        out_shape=jax.ShapeDtypeStruct((NV_PAD, C3), jnp.float32),
    )(D3, M3)


def _ki_body(p5_ref, ae2_ref, g4_ref):
    a2 = p5_ref[0][:N_E, :] + p5_ref[1][:N_E, :]
    ye2 = _guard_div(a2[:, :40], a2[:, 40:41])
    se2 = _leaky(jnp.sum(ye2 * ae2_ref[...], 1, keepdims=True))
    q2 = jnp.exp(se2 - jnp.max(se2, 0, keepdims=True))
    g4_ref[...] = jnp.concatenate(
        [ye2 * q2, q2, jnp.zeros((N_E, 7), jnp.float32)], 1)


def _edge_stage2(P5, ae2):
    return pl.pallas_call(
        _ki_body,
        out_shape=jax.ShapeDtypeStruct((N_E, C3), jnp.float32),
    )(P5, ae2)


def _kj_body(p6_ref, out_ref):
    bs = p6_ref[0][:N_V, :] + p6_ref[1][:N_V, :]
    out_ref[...] = _guard_div(bs[:, :40], bs[:, 40:41])


def _final(P6):
    return pl.pallas_call(
        _kj_body,
        out_shape=jax.ShapeDtypeStruct((N_V, N_CLASSES), jnp.float32),
    )(P6)


def _pos_encoding_const(n, channels):
    positions = jnp.arange(n, dtype=jnp.float32)[:, None]
    div1 = jnp.exp(jnp.arange(0, channels, 2, dtype=jnp.float32)
                   * (-math.log(10000.0) / channels))
    div2 = jnp.exp(jnp.arange(1, channels, 2, dtype=jnp.float32)
                   * (-math.log(10000.0) / channels))
    pe = jnp.zeros((n, channels), dtype=jnp.float32)
    pe = pe.at[:, 0::2].set(jnp.sin(positions * div1) / channels ** 0.5)
    pe = pe.at[:, 1::2].set(jnp.cos(positions * div2) / channels ** 0.5)
    return pe


def _pad_reshape(idx2, fill, kt):
    n = idx2.shape[1]
    total = 16 * kt
    pad = jnp.full((2, total - n), fill, jnp.int32)
    return jnp.concatenate([idx2, pad], 1).reshape(2, _NS, kt // _B, _B)


KT1 = 20480
KT2 = 10240


def kernel(X, v_idx, e_idx, Theta1, b1, av1, ae1, Theta2, b2, av2, ae2):
    v2 = jnp.stack([v_idx, v_idx])
    e2 = jnp.stack([e_idx, e_idx])
    off_e = jnp.array([[0], [N_E]], jnp.int32)
    comb = lambda s, d: jnp.stack([s, d], 3)
    d_v = _pad_reshape(v2, NV_PAD - 1, KT1)
    x_p2 = comb(_pad_reshape(e2 + off_e, 0, KT1), d_v)
    x_p5 = comb(_pad_reshape(v_idx.reshape(2, -1), 0, KT2),
                _pad_reshape(e_idx.reshape(2, -1), NE_PAD - 1, KT2))
    x_p6 = comb(_pad_reshape(e_idx.reshape(2, -1), 0, KT2),
                _pad_reshape(v_idx.reshape(2, -1), NV_PAD - 1, KT2))
    pe = jnp.concatenate([_pos_encoding_const(N_V, 256),
                          jnp.zeros((NV_PAD - N_V, 256), jnp.float32)])

    Th = jnp.transpose(Theta1, (1, 0, 2)).reshape(D_IN, N_HEADS * D_HID)
    Xt, sv, M = _dense1(X, Th, b1.reshape(1, -1), av1.reshape(1, -1))
    G1 = _build_g1(Xt, sv, M)

    A = _run_pass(G1, x_p5, NE_PAD)
    G2 = _edge_stage1(A, ae1)

    B = _run_pass(G2.reshape(2 * N_E, C1), x_p2, NV_PAD)
    U = _vertex_stage1(B, pe, Theta2)

    P3 = _run_pass(U, x_p5, NE_PAD)
    Ym = _edge_hspd(P3, A)
    P4 = _run_pass(Ym, x_p6, NV_PAD)

    D3, M3 = _vertex_stage2(P4, B, U,
                            b2.reshape(1, -1), av2.reshape(1, -1))
    G3 = _build_g3(D3, M3)

    P5 = _run_pass(G3, x_p5, NE_PAD)
    G4 = _edge_stage2(P5, ae2.reshape(1, -1))
    P6 = _run_pass(G4, x_p6, NV_PAD)
    return _final(P6)
